# baseline (speedup 1.0000x reference)
import functools

import jax
import jax.numpy as jnp
from jax import lax
from jax.experimental import pallas as pl
from jax.experimental.pallas import tpu as pltpu
from jax.experimental.pallas import tpu_sc as plsc

_N = 10000
_E = 320000
_D = 128
_NC = 2
_NS = 16
_NW = _NC * _NS
_EPW = _E // _NW
_C = 80
_NCHUNK = _EPW // _C
_C2 = 128
_EPWP = 10240
_NCH2 = _EPWP // _C2
_DUMP = _N
_NP = _N + 16
_BR = 16
_NB = _N // _BR
_BPT = _NB // _NS
_NBP = _NP // _BR

_mesh = plsc.VectorSubcoreMesh(core_axis_name="c", subcore_axis_name="s")


def _sc_sum(h, src_flat, dst3):
    @functools.partial(
        pl.kernel,
        out_type=jax.ShapeDtypeStruct((_NC, _NP, _D), jnp.float32),
        mesh=_mesh,
        scratch_types=[
            pltpu.VMEM((_C2,), jnp.int32),
            pltpu.VMEM((_C2,), jnp.int32),
            pltpu.VMEM((_NCH2, _C2), jnp.int32),
            pltpu.VMEM((_C2, _D), jnp.float32),
            pltpu.VMEM((_C2, _D), jnp.float32),
            pltpu.VMEM((_BR, _D), jnp.float32),
            pltpu.VMEM_SHARED((_NP, _D), jnp.float32),
            pltpu.SemaphoreType.DMA,
            pltpu.SemaphoreType.DMA,
            pltpu.SemaphoreType.DMA,
        ],
    )
    def agg(h_hbm, src_hbm, dst_hbm, psum_hbm,
            srcb0_v, srcb1_v, dst_v, rows0_v, rows1_v, zwb_v, ssum,
            sem, gsem0, gsem1):
        cid = lax.axis_index("c")
        sid = lax.axis_index("s")
        wid = cid * _NS + sid

        zero16 = jnp.zeros((16,), jnp.float32)

        @pl.loop(0, _BR)
        def _(i):
            for j in range(_D // 16):
                zwb_v[i, pl.ds(j * 16, 16)] = zero16

        lanes = lax.iota(jnp.int32, 16)
        nblk = _BPT + (sid < 2).astype(jnp.int32)

        pltpu.sync_copy(dst_hbm.at[wid], dst_v)

        @pl.loop(0, nblk)
        def _(i):
            ridx = (sid + i * _NS) * _BR + lanes
            pltpu.sync_copy(zwb_v, ssum.at[ridx])

        plsc.subcore_barrier()

        rows = (rows0_v, rows1_v)
        srcb = (srcb0_v, srcb1_v)
        gsems = (gsem0, gsem1)
        ebase = wid * _EPWP

        for b in range(2):
            pltpu.sync_copy(
                src_hbm.at[pl.ds(pl.multiple_of(ebase + b * _C2, 8), _C2)],
                srcb[b])
            pltpu.async_copy(h_hbm.at[srcb[b]], rows[b], gsems[b])

        @pl.loop(0, _NCH2, step=2)
        def _(g0):
            for b in range(2):
                g = g0 + b
                pltpu.make_async_copy(
                    h_hbm.at[srcb[b]], rows[b], gsems[b]).wait()
                pltpu.sync_copy(rows[b], ssum.at[dst_v.at[g]], add=True)

                @pl.when(g + 2 < _NCH2)
                def _():
                    off = pl.multiple_of(ebase + (g + 2) * _C2, 8)
                    pltpu.sync_copy(src_hbm.at[pl.ds(off, _C2)], srcb[b])
                    pltpu.async_copy(h_hbm.at[srcb[b]], rows[b], gsems[b])

        plsc.subcore_barrier()

        @pl.loop(0, nblk)
        def _(i):
            blk = sid + i * _NS
            ridx = blk * _BR + lanes
            pltpu.async_copy(ssum.at[ridx], zwb_v, sem).wait()
            pltpu.sync_copy(zwb_v, psum_hbm.at[cid, pl.ds(blk * _BR, _BR)])

    return agg(h, src_flat, dst3)


def _sc_count(dst):
    @functools.partial(
        pl.kernel,
        out_type=jax.ShapeDtypeStruct((_NC, _N, _D), jnp.float32),
        mesh=_mesh,
        scratch_types=[
            pltpu.VMEM((_C,), jnp.int32),
            pltpu.VMEM((_C, _D), jnp.float32),
            pltpu.VMEM((_BR, _D), jnp.float32),
            pltpu.VMEM((_BR, _D), jnp.float32),
            pltpu.VMEM_SHARED((_N, _D), jnp.float32),
            pltpu.SemaphoreType.DMA,
        ],
    )
    def cnt(dst_hbm, pcnt_hbm, dst_v, ones_v, z_v, wb_v, scnt, sem):
        cid = lax.axis_index("c")
        sid = lax.axis_index("s")
        wid = cid * _NS + sid

        zero16 = jnp.zeros((16,), jnp.float32)
        one16 = jnp.ones((16,), jnp.float32)

        @pl.loop(0, _BR)
        def _(i):
            for j in range(_D // 16):
                z_v[i, pl.ds(j * 16, 16)] = zero16

        @pl.loop(0, _C)
        def _(i):
            for j in range(_D // 16):
                ones_v[i, pl.ds(j * 16, 16)] = one16

        lanes = lax.iota(jnp.int32, 16)
        nblk = _BPT + (sid == 0).astype(jnp.int32)

        @pl.loop(0, nblk)
        def _(i):
            ridx = (sid + i * _NS) * _BR + lanes
            pltpu.sync_copy(z_v, scnt.at[ridx])

        plsc.subcore_barrier()

        ebase = wid * _EPW

        @pl.loop(0, _NCHUNK)
        def _(g):
            off = pl.multiple_of(ebase + g * _C, 8)
            pltpu.sync_copy(dst_hbm.at[pl.ds(off, _C)], dst_v)
            pltpu.sync_copy(ones_v, scnt.at[dst_v], add=True)

        plsc.subcore_barrier()

        @pl.loop(0, nblk)
        def _(i):
            blk = sid + i * _NS
            ridx = blk * _BR + lanes
            pltpu.async_copy(scnt.at[ridx], wb_v, sem).wait()
            pltpu.sync_copy(wb_v, pcnt_hbm.at[cid, pl.ds(blk * _BR, _BR)])

    return cnt(dst)


def _tc_combine(psum, pcnt, h_target, w1t, w2t, b2):
    bn = 1000
    grid = (_N // bn,)

    def body(psum_ref, pcnt_ref, ht_ref, w1t_ref, w2t_ref, b2_ref, out_ref):
        s = psum_ref[0] + psum_ref[1]
        c = pcnt_ref[0, :, 0:1] + pcnt_ref[1, :, 0:1]
        hn = s / jnp.maximum(c, 1.0)
        o = (jnp.dot(ht_ref[...], w1t_ref[...],
                     preferred_element_type=jnp.float32)
             + jnp.dot(hn, w2t_ref[...], preferred_element_type=jnp.float32)
             + b2_ref[...])
        nrm = jnp.sqrt(jnp.sum(o * o, axis=1, keepdims=True))
        out_ref[...] = o / jnp.maximum(nrm, 1e-12)

    return pl.pallas_call(
        body,
        grid=grid,
        in_specs=[
            pl.BlockSpec((_NC, bn, _D), lambda i: (0, i, 0)),
            pl.BlockSpec((_NC, bn, _D), lambda i: (0, i, 0)),
            pl.BlockSpec((bn, _D), lambda i: (i, 0)),
            pl.BlockSpec((_D, _D), lambda i: (0, 0)),
            pl.BlockSpec((_D, _D), lambda i: (0, 0)),
            pl.BlockSpec((1, _D), lambda i: (0, 0)),
        ],
        out_specs=pl.BlockSpec((bn, _D), lambda i: (i, 0)),
        out_shape=jax.ShapeDtypeStruct((_N, _D), jnp.float32),
    )(psum, pcnt, h_target, w1t, w2t, b2)


def kernel(h, h_target, edge_index, W1, W2, b2):
    dst = edge_index[0]
    src = edge_index[1]
    pad = _EPWP - _EPW
    src_flat = jnp.pad(src.reshape(_NW, _EPW), ((0, 0), (0, pad))
                       ).reshape(-1)
    dst3 = jnp.pad(dst.reshape(_NW, _EPW), ((0, 0), (0, pad)),
                   constant_values=_DUMP).reshape(_NW, _NCH2, _C2)
    psum = _sc_sum(h, src_flat, dst3)
    pcnt = _sc_count(dst)
    return _tc_combine(psum, pcnt, h_target, W1.T, W2.T, b2.reshape(1, _D))


# speedup vs baseline: 1.2031x; 1.2031x over previous
import functools

import jax
import jax.numpy as jnp
from jax import lax
from jax.experimental import pallas as pl
from jax.experimental.pallas import tpu as pltpu
from jax.experimental.pallas import tpu_sc as plsc

_N = 10000
_E = 320000
_D = 128
_NC = 2
_NS = 16
_NW = _NC * _NS
_EPW = _E // _NW
_C = 80
_NCHUNK = _EPW // _C
_BR = 16
_NB = _N // _BR
_BPT = _NB // _NS

_mesh = plsc.VectorSubcoreMesh(core_axis_name="c", subcore_axis_name="s")


def _sc_sum(h, src, dst):
    @functools.partial(
        pl.kernel,
        out_type=jax.ShapeDtypeStruct((_NC, _N, _D), jnp.float32),
        mesh=_mesh,
        scratch_types=[
            pltpu.VMEM((_C,), jnp.int32),
            pltpu.VMEM((_C,), jnp.int32),
            pltpu.VMEM((_C, _D), jnp.float32),
            pltpu.VMEM((_BR, _D), jnp.float32),
            pltpu.VMEM((_BR, _D), jnp.float32),
            pltpu.VMEM_SHARED((_N, _D), jnp.float32),
            pltpu.SemaphoreType.DMA,
        ],
    )
    def agg(h_hbm, src_hbm, dst_hbm, psum_hbm,
            src_v, dst_v, rows_v, z_v, wb_v, ssum, sem):
        cid = lax.axis_index("c")
        sid = lax.axis_index("s")
        wid = cid * _NS + sid

        zero16 = jnp.zeros((16,), jnp.float32)

        @pl.loop(0, _BR)
        def _(i):
            for j in range(_D // 16):
                z_v[i, pl.ds(j * 16, 16)] = zero16

        lanes = lax.iota(jnp.int32, 16)
        nblk = _BPT + (sid == 0).astype(jnp.int32)

        @pl.loop(0, nblk)
        def _(i):
            ridx = (sid + i * _NS) * _BR + lanes
            pltpu.sync_copy(z_v, ssum.at[ridx])

        plsc.subcore_barrier()

        ebase = wid * _EPW

        @pl.loop(0, _NCHUNK)
        def _(g):
            off = pl.multiple_of(ebase + g * _C, 8)
            pltpu.sync_copy(src_hbm.at[pl.ds(off, _C)], src_v)
            pltpu.sync_copy(dst_hbm.at[pl.ds(off, _C)], dst_v)
            pltpu.async_copy(h_hbm.at[src_v], rows_v, sem).wait()
            pltpu.sync_copy(rows_v, ssum.at[dst_v], add=True)

        plsc.subcore_barrier()

        @pl.loop(0, nblk)
        def _(i):
            blk = sid + i * _NS
            ridx = blk * _BR + lanes
            pltpu.async_copy(ssum.at[ridx], wb_v, sem).wait()
            pltpu.sync_copy(wb_v, psum_hbm.at[cid, pl.ds(blk * _BR, _BR)])

    return agg(h, src, dst)


def _sc_count(dst):
    @functools.partial(
        pl.kernel,
        out_type=jax.ShapeDtypeStruct((_NC, _N, _D), jnp.float32),
        mesh=_mesh,
        scratch_types=[
            pltpu.VMEM((_C,), jnp.int32),
            pltpu.VMEM((_C, _D), jnp.float32),
            pltpu.VMEM((_BR, _D), jnp.float32),
            pltpu.VMEM((_BR, _D), jnp.float32),
            pltpu.VMEM_SHARED((_N, _D), jnp.float32),
            pltpu.SemaphoreType.DMA,
        ],
    )
    def cnt(dst_hbm, pcnt_hbm, dst_v, ones_v, z_v, wb_v, scnt, sem):
        cid = lax.axis_index("c")
        sid = lax.axis_index("s")
        wid = cid * _NS + sid

        zero16 = jnp.zeros((16,), jnp.float32)
        one16 = jnp.ones((16,), jnp.float32)

        @pl.loop(0, _BR)
        def _(i):
            for j in range(_D // 16):
                z_v[i, pl.ds(j * 16, 16)] = zero16

        @pl.loop(0, _C)
        def _(i):
            for j in range(_D // 16):
                ones_v[i, pl.ds(j * 16, 16)] = one16

        lanes = lax.iota(jnp.int32, 16)
        nblk = _BPT + (sid == 0).astype(jnp.int32)

        @pl.loop(0, nblk)
        def _(i):
            ridx = (sid + i * _NS) * _BR + lanes
            pltpu.sync_copy(z_v, scnt.at[ridx])

        plsc.subcore_barrier()

        ebase = wid * _EPW

        @pl.loop(0, _NCHUNK)
        def _(g):
            off = pl.multiple_of(ebase + g * _C, 8)
            pltpu.sync_copy(dst_hbm.at[pl.ds(off, _C)], dst_v)
            pltpu.sync_copy(ones_v, scnt.at[dst_v], add=True)

        plsc.subcore_barrier()

        @pl.loop(0, nblk)
        def _(i):
            blk = sid + i * _NS
            ridx = blk * _BR + lanes
            pltpu.async_copy(scnt.at[ridx], wb_v, sem).wait()
            pltpu.sync_copy(wb_v, pcnt_hbm.at[cid, pl.ds(blk * _BR, _BR)])

    return cnt(dst)


def _tc_combine(psum, pcnt, h_target, w1t, w2t, b2):
    bn = 1000
    grid = (_N // bn,)

    def body(psum_ref, pcnt_ref, ht_ref, w1t_ref, w2t_ref, b2_ref, out_ref):
        s = psum_ref[0] + psum_ref[1]
        c = pcnt_ref[0, :, 0:1] + pcnt_ref[1, :, 0:1]
        hn = s / jnp.maximum(c, 1.0)
        o = (jnp.dot(ht_ref[...], w1t_ref[...],
                     preferred_element_type=jnp.float32)
             + jnp.dot(hn, w2t_ref[...], preferred_element_type=jnp.float32)
             + b2_ref[...])
        nrm = jnp.sqrt(jnp.sum(o * o, axis=1, keepdims=True))
        out_ref[...] = o / jnp.maximum(nrm, 1e-12)

    return pl.pallas_call(
        body,
        grid=grid,
        in_specs=[
            pl.BlockSpec((_NC, bn, _D), lambda i: (0, i, 0)),
            pl.BlockSpec((_NC, bn, _D), lambda i: (0, i, 0)),
            pl.BlockSpec((bn, _D), lambda i: (i, 0)),
            pl.BlockSpec((_D, _D), lambda i: (0, 0)),
            pl.BlockSpec((_D, _D), lambda i: (0, 0)),
            pl.BlockSpec((1, _D), lambda i: (0, 0)),
        ],
        out_specs=pl.BlockSpec((bn, _D), lambda i: (i, 0)),
        out_shape=jax.ShapeDtypeStruct((_N, _D), jnp.float32),
    )(psum, pcnt, h_target, w1t, w2t, b2)


def kernel(h, h_target, edge_index, W1, W2, b2):
    dst = edge_index[0]
    src = edge_index[1]
    psum = _sc_sum(h, src, dst)
    pcnt = _sc_count(dst)
    return _tc_combine(psum, pcnt, h_target, W1.T, W2.T, b2.reshape(1, _D))
